# baseline (speedup 1.0000x reference)
import jax
import jax.numpy as jnp
from jax.experimental import pallas as pl
from jax.experimental.pallas import tpu as pltpu

_N = 10000
_D = 256
_CORES = 2
_ROWS_PER_CORE = _N // _CORES
_CH = 1000
_NCH = _ROWS_PER_CORE // _CH


def _body(x_hbm, wc_ref, wn_ref, b_ref, o_hbm, x_buf, o_buf, in_sem, out_sem):
    core = pl.program_id(0)
    base = core * _ROWS_PER_CORE
    w = wc_ref[...] + wn_ref[...]
    bias = b_ref[...]

    def in_copy(i):
        return pltpu.make_async_copy(
            x_hbm.at[pl.ds(base + i * _CH, _CH), :],
            x_buf.at[i % 2],
            in_sem.at[i % 2],
        )

    def out_copy(i):
        return pltpu.make_async_copy(
            o_buf.at[i % 2],
            o_hbm.at[pl.ds(base + i * _CH, _CH), :],
            out_sem.at[i % 2],
        )

    in_copy(0).start()
    in_copy(1).start()
    for i in range(_NCH):
        in_copy(i).wait()
        if i >= 2:
            out_copy(i - 2).wait()
        o_buf[i % 2] = (
            jnp.dot(x_buf[i % 2], w, preferred_element_type=jnp.float32) + bias
        )
        out_copy(i).start()
        if i + 2 < _NCH:
            in_copy(i + 2).start()
    out_copy(_NCH - 2).wait()
    out_copy(_NCH - 1).wait()


def kernel(x, edge_index, Wc, Wn, b):
    del edge_index
    n, d_in = x.shape
    d_out = Wc.shape[1]
    b2 = b.reshape(1, d_out)
    return pl.pallas_call(
        _body,
        grid=(_CORES,),
        in_specs=[
            pl.BlockSpec(memory_space=pltpu.MemorySpace.HBM),
            pl.BlockSpec((d_in, d_out), lambda i: (0, 0)),
            pl.BlockSpec((d_in, d_out), lambda i: (0, 0)),
            pl.BlockSpec((1, d_out), lambda i: (0, 0)),
        ],
        out_specs=pl.BlockSpec(memory_space=pltpu.MemorySpace.HBM),
        out_shape=jax.ShapeDtypeStruct((n, d_out), x.dtype),
        scratch_shapes=[
            pltpu.VMEM((2, _CH, _D), jnp.float32),
            pltpu.VMEM((2, _CH, _D), jnp.float32),
            pltpu.SemaphoreType.DMA((2,)),
            pltpu.SemaphoreType.DMA((2,)),
        ],
        compiler_params=pltpu.CompilerParams(
            dimension_semantics=("parallel",),
        ),
    )(x, Wc, Wn, b2)


# speedup vs baseline: 1.4470x; 1.4470x over previous
import jax
import jax.numpy as jnp
from jax.experimental import pallas as pl
from jax.experimental.pallas import tpu as pltpu

_BM = 5000


def _fused_matmul_kernel(x_ref, w_ref, b_ref, o_ref):
    acc = jnp.dot(x_ref[...], w_ref[...], preferred_element_type=jnp.float32)
    o_ref[...] = acc + b_ref[...]


def kernel(x, edge_index, Wc, Wn, b):
    del edge_index
    n, d_in = x.shape
    d_out = Wc.shape[1]
    w = Wc + Wn
    b2 = b.reshape(1, d_out)
    return pl.pallas_call(
        _fused_matmul_kernel,
        grid=(pl.cdiv(n, _BM),),
        in_specs=[
            pl.BlockSpec((_BM, d_in), lambda i: (i, 0)),
            pl.BlockSpec((d_in, d_out), lambda i: (0, 0)),
            pl.BlockSpec((1, d_out), lambda i: (0, 0)),
        ],
        out_specs=pl.BlockSpec((_BM, d_out), lambda i: (i, 0)),
        out_shape=jax.ShapeDtypeStruct((n, d_out), x.dtype),
        compiler_params=pltpu.CompilerParams(
            dimension_semantics=("parallel",),
        ),
    )(x, w, b2)


# speedup vs baseline: 1.7967x; 1.2417x over previous
import jax
import jax.numpy as jnp
from jax.experimental import pallas as pl
from jax.experimental.pallas import tpu as pltpu

_BM = 5000


def _fused_matmul_kernel(x_ref, wc_ref, wn_ref, b_ref, o_ref):
    w = wc_ref[...] + wn_ref[...]
    acc = jnp.dot(x_ref[...], w, preferred_element_type=jnp.float32)
    o_ref[...] = acc + b_ref[...]


def kernel(x, edge_index, Wc, Wn, b):
    del edge_index
    n, d_in = x.shape
    d_out = Wc.shape[1]
    b2 = b.reshape(1, d_out)
    return pl.pallas_call(
        _fused_matmul_kernel,
        grid=(pl.cdiv(n, _BM),),
        in_specs=[
            pl.BlockSpec((_BM, d_in), lambda i: (i, 0)),
            pl.BlockSpec((d_in, d_out), lambda i: (0, 0)),
            pl.BlockSpec((d_in, d_out), lambda i: (0, 0)),
            pl.BlockSpec((1, d_out), lambda i: (0, 0)),
        ],
        out_specs=pl.BlockSpec((_BM, d_out), lambda i: (i, 0)),
        out_shape=jax.ShapeDtypeStruct((n, d_out), x.dtype),
        compiler_params=pltpu.CompilerParams(
            dimension_semantics=("parallel",),
        ),
    )(x, Wc, Wn, b2)
